# baseline (speedup 1.0000x reference)
import functools

import jax
import jax.numpy as jnp
from jax import lax
from jax.experimental import pallas as pl
from jax.experimental.pallas import tpu as pltpu
from jax.experimental.pallas import tpu_sc as plsc

BATCH = 16384
FIELDS = 26
NUM_OUTPUTS = 64

_INFO = plsc.get_sparse_core_info()
NC, NS = _INFO.num_cores, _INFO.num_subcores
NW = NC * NS

B = BATCH * FIELDS
B_PER_W = B // NW
CHUNK = 128
NCHUNK = B_PER_W // CHUNK


def _sc_gather(idx3, table):
    mesh = plsc.VectorSubcoreMesh(core_axis_name="c", subcore_axis_name="s")

    @functools.partial(
        pl.kernel,
        out_type=jax.ShapeDtypeStruct((B, NUM_OUTPUTS), jnp.float32),
        mesh=mesh,
        compiler_params=pltpu.CompilerParams(use_tc_tiling_on_sc=False),
        scratch_types=[
            pltpu.VMEM((NCHUNK, CHUNK), jnp.int32),
            pltpu.VMEM((CHUNK, NUM_OUTPUTS), jnp.float32),
            pltpu.SemaphoreType.DMA,
        ],
    )
    def k(idx_hbm, table_hbm, out_hbm, idx_v, rows_v, sem):
        wid = lax.axis_index("s") * NC + lax.axis_index("c")
        base = wid * B_PER_W
        pltpu.sync_copy(idx_hbm.at[wid], idx_v)

        def body(i, _):
            pltpu.async_copy(table_hbm.at[idx_v.at[i]], rows_v, sem).wait()
            pltpu.sync_copy(rows_v, out_hbm.at[pl.ds(base + i * CHUNK, CHUNK)])
            return 0

        lax.fori_loop(0, NCHUNK, body, 0)

    return k(idx3, table)


@jax.jit
def kernel(inputs, embed0):
    idx3 = inputs.reshape(NW, NCHUNK, CHUNK)
    flat = _sc_gather(idx3, embed0)
    return flat.reshape(BATCH, FIELDS, NUM_OUTPUTS)


# speedup vs baseline: 1.0790x; 1.0790x over previous
import functools

import jax
import jax.numpy as jnp
from jax import lax
from jax.experimental import pallas as pl
from jax.experimental.pallas import tpu as pltpu
from jax.experimental.pallas import tpu_sc as plsc

BATCH = 16384
FIELDS = 26
NUM_OUTPUTS = 64

_INFO = plsc.get_sparse_core_info()
NC, NS = _INFO.num_cores, _INFO.num_subcores
NW = NC * NS

B = BATCH * FIELDS
B_PER_W = B // NW
CHUNK = 128
NCHUNK = B_PER_W // CHUNK


NBUF = 8
DEPTH = 4


def _sc_gather(idx3, table):
    mesh = plsc.VectorSubcoreMesh(core_axis_name="c", subcore_axis_name="s")

    @functools.partial(
        pl.kernel,
        out_type=jax.ShapeDtypeStruct((B, NUM_OUTPUTS), jnp.float32),
        mesh=mesh,
        compiler_params=pltpu.CompilerParams(use_tc_tiling_on_sc=False),
        scratch_types=[
            pltpu.VMEM((NCHUNK, CHUNK), jnp.int32),
            pltpu.VMEM((NBUF, CHUNK, NUM_OUTPUTS), jnp.float32),
            pltpu.SemaphoreType.DMA,
            pltpu.SemaphoreType.DMA,
        ],
    )
    def k(idx_hbm, table_hbm, out_hbm, idx_v, rows_v, gsem, wsem):
        wid = lax.axis_index("s") * NC + lax.axis_index("c")
        base = wid * B_PER_W
        pltpu.sync_copy(idx_hbm.at[wid], idx_v)

        def g_start(chunk, slot):
            pltpu.async_copy(table_hbm.at[idx_v.at[chunk]], rows_v.at[slot], gsem)

        def g_wait(chunk, slot):
            pltpu.make_async_copy(
                table_hbm.at[idx_v.at[chunk]], rows_v.at[slot], gsem).wait()

        def out_at(chunk):
            return out_hbm.at[pl.ds(base + chunk * CHUNK, CHUNK)]

        def w_start(chunk, slot):
            pltpu.async_copy(rows_v.at[slot], out_at(chunk), wsem)

        def w_wait(chunk, slot):
            pltpu.make_async_copy(rows_v.at[slot], out_at(chunk), wsem).wait()

        for b in range(DEPTH):
            g_start(b, b)
        for i in range(DEPTH):
            g_wait(i, i)
            w_start(i, i)
            g_start(i + DEPTH, i + DEPTH)

        def body(g, _):
            for bs in range(NBUF):
                i = DEPTH + g * NBUF + bs
                slot = (DEPTH + bs) % NBUF
                g_wait(i, slot)
                w_start(i, slot)
                w_wait(i - DEPTH, bs)
                g_start(i + DEPTH, bs)
            return 0

        lax.fori_loop(0, (NCHUNK - NBUF) // NBUF, body, 0)

        for i in range(NCHUNK - DEPTH, NCHUNK):
            g_wait(i, i % NBUF)
            w_start(i, i % NBUF)
        for i in range(NCHUNK - NBUF, NCHUNK):
            w_wait(i, i % NBUF)

    return k(idx3, table)


@jax.jit
def kernel(inputs, embed0):
    idx3 = inputs.reshape(NW, NCHUNK, CHUNK)
    flat = _sc_gather(idx3, embed0)
    return flat.reshape(BATCH, FIELDS, NUM_OUTPUTS)


# speedup vs baseline: 1.0827x; 1.0034x over previous
import functools

import jax
import jax.numpy as jnp
from jax import lax
from jax.experimental import pallas as pl
from jax.experimental.pallas import tpu as pltpu
from jax.experimental.pallas import tpu_sc as plsc

BATCH = 16384
FIELDS = 26
NUM_OUTPUTS = 64

_INFO = plsc.get_sparse_core_info()
NC, NS = _INFO.num_cores, _INFO.num_subcores
NW = NC * NS

B = BATCH * FIELDS
B_PER_W = B // NW
CHUNK = 256
NCHUNK = B_PER_W // CHUNK


NBUF = 4
DEPTH = 2


def _sc_gather(idx3, table):
    mesh = plsc.VectorSubcoreMesh(core_axis_name="c", subcore_axis_name="s")

    @functools.partial(
        pl.kernel,
        out_type=jax.ShapeDtypeStruct((B, NUM_OUTPUTS), jnp.float32),
        mesh=mesh,
        compiler_params=pltpu.CompilerParams(use_tc_tiling_on_sc=False),
        scratch_types=[
            pltpu.VMEM((NCHUNK, CHUNK), jnp.int32),
            pltpu.VMEM((NBUF, CHUNK, NUM_OUTPUTS), jnp.float32),
            pltpu.SemaphoreType.DMA,
            pltpu.SemaphoreType.DMA,
        ],
    )
    def k(idx_hbm, table_hbm, out_hbm, idx_v, rows_v, gsem, wsem):
        wid = lax.axis_index("s") * NC + lax.axis_index("c")
        base = wid * B_PER_W
        pltpu.sync_copy(idx_hbm.at[wid], idx_v)

        def g_start(chunk, slot):
            pltpu.async_copy(table_hbm.at[idx_v.at[chunk]], rows_v.at[slot], gsem)

        def g_wait(chunk, slot):
            pltpu.make_async_copy(
                table_hbm.at[idx_v.at[chunk]], rows_v.at[slot], gsem).wait()

        def out_at(chunk):
            return out_hbm.at[pl.ds(base + chunk * CHUNK, CHUNK)]

        def w_start(chunk, slot):
            pltpu.async_copy(rows_v.at[slot], out_at(chunk), wsem)

        def w_wait(chunk, slot):
            pltpu.make_async_copy(rows_v.at[slot], out_at(chunk), wsem).wait()

        for b in range(DEPTH):
            g_start(b, b)
        for i in range(DEPTH):
            g_wait(i, i)
            w_start(i, i)
            g_start(i + DEPTH, i + DEPTH)

        def body(g, _):
            for bs in range(NBUF):
                i = DEPTH + g * NBUF + bs
                slot = (DEPTH + bs) % NBUF
                g_wait(i, slot)
                w_start(i, slot)
                w_wait(i - DEPTH, bs)
                g_start(i + DEPTH, bs)
            return 0

        lax.fori_loop(0, (NCHUNK - NBUF) // NBUF, body, 0)

        for i in range(NCHUNK - DEPTH, NCHUNK):
            g_wait(i, i % NBUF)
            w_start(i, i % NBUF)
        for i in range(NCHUNK - NBUF, NCHUNK):
            w_wait(i, i % NBUF)

    return k(idx3, table)


@jax.jit
def kernel(inputs, embed0):
    idx3 = inputs.reshape(NW, NCHUNK, CHUNK)
    flat = _sc_gather(idx3, embed0)
    return flat.reshape(BATCH, FIELDS, NUM_OUTPUTS)
